# baseline (speedup 1.0000x reference)
import functools

import jax
import jax.numpy as jnp
from jax import lax
from jax.experimental import pallas as pl
from jax.experimental.pallas import tpu as pltpu
from jax.experimental.pallas import tpu_sc as plsc

B, L, D = 4, 2048, 1024
SC_BATCHES = 1
TC_B = B - SC_BATCHES
NC, NS = 2, 16
NW = NC * NS
SC_ROWS = SC_BATCHES * L
ROWS_W = SC_ROWS // NW
CH = 16
CHW = CH * D
SC_BASE = TC_B * L


def _tc_body(x_ref, e_ref, o_ref):
    o_ref[...] = x_ref[...] + e_ref[...]


def _tc_add(inputs, emb3):
    BL = 512
    return pl.pallas_call(
        _tc_body,
        grid=(L // BL,),
        in_specs=[
            pl.BlockSpec((TC_B, BL, D), lambda j: (0, j, 0)),
            pl.BlockSpec((1, BL, D), lambda j: (0, j, 0)),
        ],
        out_specs=pl.BlockSpec((TC_B, BL, D), lambda j: (0, j, 0)),
        out_shape=jax.ShapeDtypeStruct((TC_B, L, D), inputs.dtype),
    )(inputs, emb3)


@functools.partial(
    pl.kernel,
    out_type=jax.ShapeDtypeStruct((SC_ROWS * D,), jnp.float32),
    mesh=plsc.VectorSubcoreMesh(core_axis_name="c", subcore_axis_name="s"),
    scratch_types=[
        pltpu.VMEM((CHW,), jnp.float32),
        pltpu.VMEM((CHW,), jnp.float32),
    ],
)
def _sc_add(in_hbm, emb_hbm, out_hbm, xbuf, ebuf):
    wid = lax.axis_index("s") * NC + lax.axis_index("c")
    rbase = wid * ROWS_W

    @pl.loop(0, ROWS_W // CH)
    def _chunk(ci):
        row = rbase + ci * CH
        pltpu.sync_copy(in_hbm.at[pl.ds((SC_BASE + row) * D, CHW)], xbuf)
        pltpu.sync_copy(emb_hbm.at[pl.ds(row * D, CHW)], ebuf)

        @pl.loop(0, CHW // 16, unroll=8)
        def _add(i):
            s = pl.ds(i * 16, 16)
            xbuf[s] = xbuf[s] + ebuf[s]

        pltpu.sync_copy(xbuf, out_hbm.at[pl.ds(row * D, CHW)])


def kernel(inputs, emb_table):
    emb3 = emb_table[None]
    tc_out = _tc_add(inputs[:TC_B], emb3)
    sc_out = _sc_add(inputs.reshape(-1), emb_table.reshape(-1))
    return jnp.concatenate([tc_out, sc_out.reshape(SC_BATCHES, L, D)], axis=0)


# speedup vs baseline: 5.3735x; 5.3735x over previous
import functools

import jax
import jax.numpy as jnp
from jax import lax
from jax.experimental import pallas as pl
from jax.experimental.pallas import tpu as pltpu
from jax.experimental.pallas import tpu_sc as plsc

B, L, D = 4, 2048, 1024
SC_BATCHES = 1
TC_B = B - SC_BATCHES
NC, NS = 2, 16
NW = NC * NS
SC_ROWS = SC_BATCHES * L
ROWS_W = SC_ROWS // NW
CH = 16
CHW = CH * D
SC_BASE = TC_B * L


def _tc_body(x_ref, e_ref, o_ref):
    o_ref[...] = x_ref[...] + e_ref[...]


def _tc_add(inputs, emb3):
    BL = 512
    return pl.pallas_call(
        _tc_body,
        grid=(L // BL,),
        in_specs=[
            pl.BlockSpec((TC_B, BL, D), lambda j: (0, j, 0)),
            pl.BlockSpec((1, BL, D), lambda j: (0, j, 0)),
        ],
        out_specs=pl.BlockSpec((TC_B, BL, D), lambda j: (0, j, 0)),
        out_shape=jax.ShapeDtypeStruct((TC_B, L, D), inputs.dtype),
    )(inputs, emb3)


@functools.partial(
    pl.kernel,
    out_type=jax.ShapeDtypeStruct((SC_ROWS * D,), jnp.float32),
    mesh=plsc.VectorSubcoreMesh(core_axis_name="c", subcore_axis_name="s"),
    scratch_types=[
        pltpu.VMEM((CHW,), jnp.float32),
        pltpu.VMEM((CHW,), jnp.float32),
    ],
)
def _sc_add(in_hbm, emb_hbm, out_hbm, xbuf, ebuf):
    wid = lax.axis_index("s") * NC + lax.axis_index("c")
    rbase = wid * ROWS_W

    @pl.loop(0, ROWS_W // CH)
    def _chunk(ci):
        row = rbase + ci * CH
        pltpu.sync_copy(in_hbm.at[pl.ds((SC_BASE + row) * D, CHW)], xbuf)
        pltpu.sync_copy(emb_hbm.at[pl.ds(row * D, CHW)], ebuf)

        @pl.loop(0, CHW // 16, unroll=8)
        def _add(i):
            s = pl.ds(i * 16, 16)
            xbuf[s] = xbuf[s] + ebuf[s]

        pltpu.sync_copy(xbuf, out_hbm.at[pl.ds(row * D, CHW)])


def _tc_full(inputs, emb3, BL=256):
    return pl.pallas_call(
        _tc_body,
        grid=(L // BL,),
        in_specs=[
            pl.BlockSpec((B, BL, D), lambda j: (0, j, 0)),
            pl.BlockSpec((1, BL, D), lambda j: (0, j, 0)),
        ],
        out_specs=pl.BlockSpec((B, BL, D), lambda j: (0, j, 0)),
        out_shape=jax.ShapeDtypeStruct((B, L, D), inputs.dtype),
    )(inputs, emb3)


def kernel(inputs, emb_table):
    return _tc_full(inputs, emb_table[None])
